# initial kernel scaffold (unmeasured)
import functools

import jax
import jax.numpy as jnp
from jax import lax
from jax.experimental import pallas as pl
from jax.experimental.pallas import tpu as pltpu

N_DEV = 8
M = 1024
D = 1024

_signal = getattr(pl, "semaphore_signal", None) or pltpu.semaphore_signal
_swait = getattr(pl, "semaphore_wait", None) or pltpu.semaphore_wait


def kernel(partial, gamma):
    x = jnp.reshape(partial, (N_DEV * M, D))
    g2 = jnp.reshape(gamma, (1, D))

    def body(x_ref, g_ref, out_ref, comm_ref, stage_ref,
             stage_sems, send_sems, recv_sems):
        d = lax.axis_index("i")
        left = (d + N_DEV - 1) % N_DEV
        right = (d + 1) % N_DEV

        c0 = (d + N_DEV - 1) % N_DEV
        init = pltpu.make_async_copy(
            x_ref.at[pl.ds(c0 * M, M), :], comm_ref.at[N_DEV - 1],
            stage_sems.at[2],
        )
        init.start()

        barrier_sem = pltpu.get_barrier_semaphore()
        for nbr in (left, right):
            _signal(barrier_sem, inc=1, device_id=(nbr,),
                    device_id_type=pl.DeviceIdType.MESH)
        _swait(barrier_sem, 2)
        init.wait()

        for h in range(N_DEV - 1):
            src = comm_ref.at[N_DEV - 1] if h == 0 else comm_ref.at[h - 1]
            rdma = pltpu.make_async_remote_copy(
                src_ref=src,
                dst_ref=comm_ref.at[h],
                send_sem=send_sems.at[h],
                recv_sem=recv_sems.at[h],
                device_id=(right,),
                device_id_type=pl.DeviceIdType.MESH,
            )
            rdma.start()
            c = (d + 2 * N_DEV - h - 2) % N_DEV
            st = pltpu.make_async_copy(
                x_ref.at[pl.ds(c * M, M), :], stage_ref.at[h % 2],
                stage_sems.at[h % 2],
            )
            st.start()
            rdma.wait()
            st.wait()
            comm_ref[h] = comm_ref[h] + stage_ref[h % 2]

        y = comm_ref[N_DEV - 2]
        ms = jnp.sum(y * y, axis=-1, keepdims=True) * (1.0 / D)
        out_ref[...] = y * lax.rsqrt(ms + 1e-6) * g_ref[...]

        @functools.partial(pl.run_scoped, sem2=pltpu.SemaphoreType.REGULAR)
        def _(sem2):
            for nbr in (left, right):
                _signal(sem2, inc=1, device_id=(nbr,),
                        device_id_type=pl.DeviceIdType.MESH)
            _swait(sem2, 2)

    return pl.pallas_call(
        body,
        out_shape=jax.ShapeDtypeStruct((M, D), jnp.float32),
        in_specs=[
            pl.BlockSpec(memory_space=pltpu.ANY),
            pl.BlockSpec(memory_space=pltpu.VMEM),
        ],
        out_specs=pl.BlockSpec(memory_space=pltpu.VMEM),
        scratch_shapes=[
            pltpu.VMEM((N_DEV, M, D), jnp.float32),
            pltpu.VMEM((2, M, D), jnp.float32),
            pltpu.SemaphoreType.DMA((3,)),
            pltpu.SemaphoreType.DMA((N_DEV - 1,)),
            pltpu.SemaphoreType.DMA((N_DEV - 1,)),
        ],
        compiler_params=pltpu.CompilerParams(collective_id=0),
    )(x, g2)


# baseline (device time: 343804 ns/iter reference)
import functools

import jax
import jax.numpy as jnp
from jax import lax
from jax.experimental import pallas as pl
from jax.experimental.pallas import tpu as pltpu

N_DEV = 8
M = 1024
D = 1024

_signal = getattr(pl, "semaphore_signal", None) or pltpu.semaphore_signal
_swait = getattr(pl, "semaphore_wait", None) or pltpu.semaphore_wait


def kernel(partial, gamma):
    x = jnp.reshape(partial, (N_DEV * M, D))
    g2 = jnp.reshape(gamma, (1, D))

    def body(x_ref, g_ref, out_ref, comm_ref, stage_ref,
             stage_sems, send_sems, recv_sems):
        d = lax.axis_index("i")
        left = (d + N_DEV - 1) % N_DEV
        right = (d + 1) % N_DEV

        c0 = (d + N_DEV - 1) % N_DEV
        init = pltpu.make_async_copy(
            x_ref.at[pl.ds(c0 * M, M), :], comm_ref.at[N_DEV - 1],
            stage_sems.at[2],
        )
        init.start()

        barrier_sem = pltpu.get_barrier_semaphore()
        for nbr in (left, right):
            _signal(barrier_sem, inc=1, device_id=(nbr,),
                    device_id_type=pl.DeviceIdType.MESH)
        _swait(barrier_sem, 2)
        init.wait()

        for h in range(N_DEV - 1):
            src = comm_ref.at[N_DEV - 1] if h == 0 else comm_ref.at[h - 1]
            rdma = pltpu.make_async_remote_copy(
                src_ref=src,
                dst_ref=comm_ref.at[h],
                send_sem=send_sems.at[h],
                recv_sem=recv_sems.at[h],
                device_id=(right,),
                device_id_type=pl.DeviceIdType.MESH,
            )
            rdma.start()
            c = (d + 2 * N_DEV - h - 2) % N_DEV
            st = pltpu.make_async_copy(
                x_ref.at[pl.ds(c * M, M), :], stage_ref.at[h % 2],
                stage_sems.at[h % 2],
            )
            st.start()
            rdma.wait()
            st.wait()
            comm_ref[h] = comm_ref[h] + stage_ref[h % 2]

        y = comm_ref[N_DEV - 2]
        ms = jnp.sum(y * y, axis=-1, keepdims=True) * (1.0 / D)
        out_ref[...] = y * lax.rsqrt(ms + 1e-6) * g_ref[...]

        @functools.partial(pl.run_scoped, sem2=pltpu.SemaphoreType.REGULAR)
        def _(sem2):
            for nbr in (left, right):
                _signal(sem2, inc=1, device_id=(nbr,),
                        device_id_type=pl.DeviceIdType.MESH)
            _swait(sem2, 2)

    return pl.pallas_call(
        body,
        out_shape=jax.ShapeDtypeStruct((M, D), jnp.float32),
        in_specs=[
            pl.BlockSpec(memory_space=pl.ANY),
            pl.BlockSpec(memory_space=pltpu.VMEM),
        ],
        out_specs=pl.BlockSpec(memory_space=pltpu.VMEM),
        scratch_shapes=[
            pltpu.VMEM((N_DEV, M, D), jnp.float32),
            pltpu.VMEM((2, M, D), jnp.float32),
            pltpu.SemaphoreType.DMA((3,)),
            pltpu.SemaphoreType.DMA((N_DEV - 1,)),
            pltpu.SemaphoreType.DMA((N_DEV - 1,)),
        ],
        compiler_params=pltpu.CompilerParams(
            collective_id=0, vmem_limit_bytes=60 * 1024 * 1024,
        ),
    )(x, g2)


# device time: 189005 ns/iter; 1.8190x vs baseline; 1.8190x over previous
import functools

import jax
import jax.numpy as jnp
from jax import lax
from jax.experimental import pallas as pl
from jax.experimental.pallas import tpu as pltpu

N_DEV = 8
M = 1024
H = M // 2
D = 1024

_signal = getattr(pl, "semaphore_signal", None) or pltpu.semaphore_signal
_swait = getattr(pl, "semaphore_wait", None) or pltpu.semaphore_wait


def kernel(partial, gamma):
    x = jnp.reshape(partial, (N_DEV * M, D))
    g2 = jnp.reshape(gamma, (1, D))

    def body(x_ref, g_ref, out_ref, cw_ref, ccw_ref, stage_ref,
             stage_sems, send_cw, recv_cw, send_ccw, recv_ccw):
        d = lax.axis_index("i")
        left = (d + N_DEV - 1) % N_DEV
        right = (d + 1) % N_DEV

        c_cw0 = (d + N_DEV - 1) % N_DEV
        c_ccw0 = (d + 1) % N_DEV
        init_cw = pltpu.make_async_copy(
            x_ref.at[pl.ds(c_cw0 * M, H), :], cw_ref.at[N_DEV - 1],
            stage_sems.at[4],
        )
        init_ccw = pltpu.make_async_copy(
            x_ref.at[pl.ds(c_ccw0 * M + H, H), :], ccw_ref.at[N_DEV - 1],
            stage_sems.at[5],
        )
        init_cw.start()
        init_ccw.start()

        barrier_sem = pltpu.get_barrier_semaphore()
        for nbr in (left, right):
            _signal(barrier_sem, inc=1, device_id=(nbr,),
                    device_id_type=pl.DeviceIdType.MESH)
        _swait(barrier_sem, 2)
        init_cw.wait()
        init_ccw.wait()

        for h in range(N_DEV - 1):
            src_cw = cw_ref.at[N_DEV - 1] if h == 0 else cw_ref.at[h - 1]
            src_ccw = ccw_ref.at[N_DEV - 1] if h == 0 else ccw_ref.at[h - 1]
            rdma_cw = pltpu.make_async_remote_copy(
                src_ref=src_cw,
                dst_ref=cw_ref.at[h],
                send_sem=send_cw.at[h],
                recv_sem=recv_cw.at[h],
                device_id=(right,),
                device_id_type=pl.DeviceIdType.MESH,
            )
            rdma_ccw = pltpu.make_async_remote_copy(
                src_ref=src_ccw,
                dst_ref=ccw_ref.at[h],
                send_sem=send_ccw.at[h],
                recv_sem=recv_ccw.at[h],
                device_id=(left,),
                device_id_type=pl.DeviceIdType.MESH,
            )
            rdma_cw.start()
            rdma_ccw.start()

            c_cw = (d + 2 * N_DEV - h - 2) % N_DEV
            c_ccw = (d + h + 2) % N_DEV
            st_cw = pltpu.make_async_copy(
                x_ref.at[pl.ds(c_cw * M, H), :],
                stage_ref.at[2 * (h % 2)],
                stage_sems.at[2 * (h % 2)],
            )
            st_ccw = pltpu.make_async_copy(
                x_ref.at[pl.ds(c_ccw * M + H, H), :],
                stage_ref.at[2 * (h % 2) + 1],
                stage_sems.at[2 * (h % 2) + 1],
            )
            st_cw.start()
            st_ccw.start()

            rdma_cw.wait()
            st_cw.wait()
            cw_ref[h] = cw_ref[h] + stage_ref[2 * (h % 2)]
            rdma_ccw.wait()
            st_ccw.wait()
            ccw_ref[h] = ccw_ref[h] + stage_ref[2 * (h % 2) + 1]

        g = g_ref[...]
        y_t = cw_ref[N_DEV - 2]
        ms_t = jnp.sum(y_t * y_t, axis=-1, keepdims=True) * (1.0 / D)
        out_ref[pl.ds(0, H), :] = y_t * lax.rsqrt(ms_t + 1e-6) * g
        y_b = ccw_ref[N_DEV - 2]
        ms_b = jnp.sum(y_b * y_b, axis=-1, keepdims=True) * (1.0 / D)
        out_ref[pl.ds(H, H), :] = y_b * lax.rsqrt(ms_b + 1e-6) * g

        @functools.partial(pl.run_scoped, sem2=pltpu.SemaphoreType.REGULAR)
        def _(sem2):
            for nbr in (left, right):
                _signal(sem2, inc=1, device_id=(nbr,),
                        device_id_type=pl.DeviceIdType.MESH)
            _swait(sem2, 2)

    return pl.pallas_call(
        body,
        out_shape=jax.ShapeDtypeStruct((M, D), jnp.float32),
        in_specs=[
            pl.BlockSpec(memory_space=pl.ANY),
            pl.BlockSpec(memory_space=pltpu.VMEM),
        ],
        out_specs=pl.BlockSpec(memory_space=pltpu.VMEM),
        scratch_shapes=[
            pltpu.VMEM((N_DEV, H, D), jnp.float32),
            pltpu.VMEM((N_DEV, H, D), jnp.float32),
            pltpu.VMEM((4, H, D), jnp.float32),
            pltpu.SemaphoreType.DMA((6,)),
            pltpu.SemaphoreType.DMA((N_DEV - 1,)),
            pltpu.SemaphoreType.DMA((N_DEV - 1,)),
            pltpu.SemaphoreType.DMA((N_DEV - 1,)),
            pltpu.SemaphoreType.DMA((N_DEV - 1,)),
        ],
        compiler_params=pltpu.CompilerParams(
            collective_id=0, vmem_limit_bytes=60 * 1024 * 1024,
        ),
    )(x, g2)


# device time: 175748 ns/iter; 1.9562x vs baseline; 1.0754x over previous
import functools

import jax
import jax.numpy as jnp
from jax import lax
from jax.experimental import pallas as pl
from jax.experimental.pallas import tpu as pltpu

N_DEV = 8
M = 1024
Q = M // 4
D = 1024
N_HOP = N_DEV - 1

_signal = getattr(pl, "semaphore_signal", None) or pltpu.semaphore_signal
_swait = getattr(pl, "semaphore_wait", None) or pltpu.semaphore_wait


def kernel(partial, gamma):
    x = jnp.reshape(partial, (N_DEV * M, D))
    g2 = jnp.reshape(gamma, (1, D))

    def body(x_ref, g_ref, out_ref, cw_ref, ccw_ref, stage_ref,
             init_sems, stage_sems, send_cw, recv_cw, send_ccw, recv_ccw):
        d = lax.axis_index("i")
        left = (d + N_DEV - 1) % N_DEV
        right = (d + 1) % N_DEV

        def comm(k):
            return cw_ref if k < 2 else ccw_ref

        def sems(k):
            return (send_cw, recv_cw) if k < 2 else (send_ccw, recv_ccw)

        def chunk_idx(k, h):
            if k < 2:
                return (d + 2 * N_DEV - h - 2) % N_DEV
            return (d + h + 2) % N_DEV

        def x_rows(k, c):
            return x_ref.at[pl.ds(c * M + (k % 2) * Q + (k // 2) * 2 * Q, Q), :]

        def stream(k):
            return k % 2

        def rdma(k, h):
            s = stream(k)
            send_s, recv_s = sems(k)
            src = comm(k).at[s, N_DEV - 1] if h == 0 else comm(k).at[s, h - 1]
            return pltpu.make_async_remote_copy(
                src_ref=src,
                dst_ref=comm(k).at[s, h],
                send_sem=send_s.at[s, h],
                recv_sem=recv_s.at[s, h],
                device_id=((right,) if k < 2 else (left,)),
                device_id_type=pl.DeviceIdType.MESH,
            )

        def stage_copy(k, h):
            p = h % 2
            return pltpu.make_async_copy(
                x_rows(k, chunk_idx(k, h)),
                stage_ref.at[p * 4 + k],
                stage_sems.at[p * 4 + k],
            )

        inits = []
        for k in range(4):
            ic = pltpu.make_async_copy(
                x_rows(k, chunk_idx(k, -1)),
                comm(k).at[stream(k), N_DEV - 1],
                init_sems.at[k],
            )
            ic.start()
            inits.append(ic)

        barrier_sem = pltpu.get_barrier_semaphore()
        for nbr in (left, right):
            _signal(barrier_sem, inc=1, device_id=(nbr,),
                    device_id_type=pl.DeviceIdType.MESH)
        _swait(barrier_sem, 2)
        for k in range(4):
            inits[k].wait()
            rdma(k, 0).start()
            stage_copy(k, 0).start()

        for h in range(N_HOP):
            for k in range(4):
                s = stream(k)
                rdma(k, h).wait_recv()
                stage_copy(k, h).wait()
                comm(k)[s, h] = comm(k)[s, h] + stage_ref[(h % 2) * 4 + k]
                if h < N_HOP - 1:
                    rdma(k, h + 1).start()
            if h < N_HOP - 1:
                for k in range(4):
                    stage_copy(k, h + 1).start()

        g = g_ref[...]
        for k in range(4):
            y = comm(k)[stream(k), N_DEV - 2]
            ms = jnp.sum(y * y, axis=-1, keepdims=True) * (1.0 / D)
            r0 = (k % 2) * Q + (k // 2) * 2 * Q
            out_ref[pl.ds(r0, Q), :] = y * lax.rsqrt(ms + 1e-6) * g

        for h in range(N_HOP):
            for k in range(4):
                rdma(k, h).wait_send()

        @functools.partial(pl.run_scoped, sem2=pltpu.SemaphoreType.REGULAR)
        def _(sem2):
            for nbr in (left, right):
                _signal(sem2, inc=1, device_id=(nbr,),
                        device_id_type=pl.DeviceIdType.MESH)
            _swait(sem2, 2)

    return pl.pallas_call(
        body,
        out_shape=jax.ShapeDtypeStruct((M, D), jnp.float32),
        in_specs=[
            pl.BlockSpec(memory_space=pl.ANY),
            pl.BlockSpec(memory_space=pltpu.VMEM),
        ],
        out_specs=pl.BlockSpec(memory_space=pltpu.VMEM),
        scratch_shapes=[
            pltpu.VMEM((2, N_DEV, Q, D), jnp.float32),
            pltpu.VMEM((2, N_DEV, Q, D), jnp.float32),
            pltpu.VMEM((8, Q, D), jnp.float32),
            pltpu.SemaphoreType.DMA((4,)),
            pltpu.SemaphoreType.DMA((8,)),
            pltpu.SemaphoreType.DMA((2, N_HOP)),
            pltpu.SemaphoreType.DMA((2, N_HOP)),
            pltpu.SemaphoreType.DMA((2, N_HOP)),
            pltpu.SemaphoreType.DMA((2, N_HOP)),
        ],
        compiler_params=pltpu.CompilerParams(
            collective_id=0, vmem_limit_bytes=60 * 1024 * 1024,
        ),
    )(x, g2)


# device time: 175363 ns/iter; 1.9605x vs baseline; 1.0022x over previous
import functools

import jax
import jax.numpy as jnp
from jax import lax
from jax.experimental import pallas as pl
from jax.experimental.pallas import tpu as pltpu

N_DEV = 8
M = 1024
Q = M // 4
D = 1024
N_HOP = N_DEV - 1

_signal = getattr(pl, "semaphore_signal", None) or pltpu.semaphore_signal
_swait = getattr(pl, "semaphore_wait", None) or pltpu.semaphore_wait


def kernel(partial, gamma):
    x = jnp.reshape(partial, (N_DEV * M, D))
    g2 = jnp.reshape(gamma, (1, D))

    def body(x_ref, g_ref, out_ref, cw_ref, ccw_ref, stage_ref,
             init_sems, stage_sems, send_cw, recv_cw, send_ccw, recv_ccw):
        d = lax.axis_index("i")
        left = (d + N_DEV - 1) % N_DEV
        right = (d + 1) % N_DEV

        def comm(k):
            return cw_ref if k < 2 else ccw_ref

        def sems(k):
            return (send_cw, recv_cw) if k < 2 else (send_ccw, recv_ccw)

        def chunk_idx(k, h):
            if k < 2:
                return (d + 2 * N_DEV - h - 2) % N_DEV
            return (d + h + 2) % N_DEV

        def x_rows(k, c):
            return x_ref.at[pl.ds(c * M + (k % 2) * Q + (k // 2) * 2 * Q, Q), :]

        def stream(k):
            return k % 2

        def rdma(k, h):
            s = stream(k)
            send_s, recv_s = sems(k)
            src = comm(k).at[s, N_DEV - 1] if h == 0 else comm(k).at[s, h - 1]
            return pltpu.make_async_remote_copy(
                src_ref=src,
                dst_ref=comm(k).at[s, h],
                send_sem=send_s.at[s, h],
                recv_sem=recv_s.at[s, h],
                device_id=((right,) if k < 2 else (left,)),
                device_id_type=pl.DeviceIdType.MESH,
            )

        def stage_copy(k, h):
            p = h % 2
            return pltpu.make_async_copy(
                x_rows(k, chunk_idx(k, h)),
                stage_ref.at[p * 4 + k],
                stage_sems.at[p * 4 + k],
            )

        inits = []
        for k in range(4):
            ic = pltpu.make_async_copy(
                x_rows(k, chunk_idx(k, -1)),
                comm(k).at[stream(k), N_DEV - 1],
                init_sems.at[k],
            )
            ic.start()
            inits.append(ic)

        barrier_sem = pltpu.get_barrier_semaphore()
        for nbr in (left, right):
            _signal(barrier_sem, inc=1, device_id=(nbr,),
                    device_id_type=pl.DeviceIdType.MESH)
        _swait(barrier_sem, 2)
        for k in range(4):
            inits[k].wait()
            rdma(k, 0).start()
            stage_copy(k, 0).start()

        g = g_ref[...]
        for h in range(N_HOP):
            for k in range(4):
                s = stream(k)
                rdma(k, h).wait_recv()
                stage_copy(k, h).wait()
                if h < N_HOP - 1:
                    comm(k)[s, h] = comm(k)[s, h] + stage_ref[(h % 2) * 4 + k]
                    rdma(k, h + 1).start()
                else:
                    y = comm(k)[s, h] + stage_ref[(h % 2) * 4 + k]
                    ms = jnp.sum(y * y, axis=-1, keepdims=True) * (1.0 / D)
                    r0 = (k % 2) * Q + (k // 2) * 2 * Q
                    out_ref[pl.ds(r0, Q), :] = y * lax.rsqrt(ms + 1e-6) * g
            if h < N_HOP - 1:
                for k in range(4):
                    stage_copy(k, h + 1).start()

        for h in range(N_HOP):
            for k in range(4):
                rdma(k, h).wait_send()

        @functools.partial(pl.run_scoped, sem2=pltpu.SemaphoreType.REGULAR)
        def _(sem2):
            for nbr in (left, right):
                _signal(sem2, inc=1, device_id=(nbr,),
                        device_id_type=pl.DeviceIdType.MESH)
            _swait(sem2, 2)

    return pl.pallas_call(
        body,
        out_shape=jax.ShapeDtypeStruct((M, D), jnp.float32),
        in_specs=[
            pl.BlockSpec(memory_space=pl.ANY),
            pl.BlockSpec(memory_space=pltpu.VMEM),
        ],
        out_specs=pl.BlockSpec(memory_space=pltpu.VMEM),
        scratch_shapes=[
            pltpu.VMEM((2, N_DEV, Q, D), jnp.float32),
            pltpu.VMEM((2, N_DEV, Q, D), jnp.float32),
            pltpu.VMEM((8, Q, D), jnp.float32),
            pltpu.SemaphoreType.DMA((4,)),
            pltpu.SemaphoreType.DMA((8,)),
            pltpu.SemaphoreType.DMA((2, N_HOP)),
            pltpu.SemaphoreType.DMA((2, N_HOP)),
            pltpu.SemaphoreType.DMA((2, N_HOP)),
            pltpu.SemaphoreType.DMA((2, N_HOP)),
        ],
        compiler_params=pltpu.CompilerParams(
            collective_id=0, vmem_limit_bytes=60 * 1024 * 1024,
        ),
    )(x, g2)
